# baseline (speedup 1.0000x reference)
import functools
import numpy as np
import jax
import jax.numpy as jnp
from jax import lax
from jax.experimental import pallas as pl
from jax.experimental.pallas import tpu as pltpu
from jax.experimental.pallas import tpu_sc as plsc

N_HEADS = 8
N_POINTS = 4
B, C, H, W = 8, 768, 32, 32
DH = C // N_HEADS
N = H * W


def _stage1_body(q_ref, k_ref, wt_ref, b_ref, idx_ref, w_ref, kt_ref):
    o = jnp.dot(wt_ref[...], q_ref[...], preferred_element_type=jnp.float32,
                precision=jax.lax.Precision.HIGHEST)
    o = o + b_ref[...]
    ox = o[0:32].reshape(N_HEADS, N_POINTS, N)
    oy = o[32:64].reshape(N_HEADS, N_POINTS, N)
    s = o[64:96].reshape(N_HEADS, N_POINTS, N)

    m = jnp.max(s, axis=1, keepdims=True)
    e = jnp.exp(s - m)
    wsm = e / jnp.sum(e, axis=1, keepdims=True)

    n = jax.lax.broadcasted_iota(jnp.int32, (N_HEADS, N_POINTS, N), 2)
    xpix = (n & (W - 1)).astype(jnp.float32)
    ypix = (n >> 5).astype(jnp.float32)
    gx = -1.0 + xpix * (2.0 / (W - 1))
    gy = -1.0 + ypix * (2.0 / (H - 1))

    px = (gx + ox * 0.1 + 1.0) * ((W - 1) / 2.0)
    py = (gy + oy * 0.1 + 1.0) * ((H - 1) / 2.0)
    x0f = jnp.floor(px)
    y0f = jnp.floor(py)
    x0 = x0f.astype(jnp.int32)
    y0 = y0f.astype(jnp.int32)
    x1 = x0 + 1
    y1 = y0 + 1
    wx1 = px - x0f
    wx0 = 1.0 - wx1
    wy1 = py - y0f
    wy0 = 1.0 - wy1

    def corner(xc, yc, wxc, wyc):
        valid = ((xc >= 0) & (xc < W) & (yc >= 0) & (yc < H)).astype(jnp.float32)
        idx = jnp.clip(yc, 0, H - 1) * W + jnp.clip(xc, 0, W - 1)
        w = wsm * wyc * wxc * valid
        return idx, w

    i00, w00 = corner(x0, y0, wx0, wy0)
    i10, w10 = corner(x1, y0, wx1, wy0)
    i01, w01 = corner(x0, y1, wx0, wy1)
    i11, w11 = corner(x1, y1, wx1, wy1)
    idx_ref[...] = jnp.concatenate([i00, i10, i01, i11], axis=1)
    w_ref[...] = jnp.concatenate([w00, w10, w01, w11], axis=1)

    kt = k_ref[...].reshape(N_HEADS, DH, N)
    kt_ref[...] = jnp.swapaxes(kt, 1, 2)


def _stage1(query, key_feat, Wt, b_offp):
    return pl.pallas_call(
        _stage1_body,
        grid=(B,),
        in_specs=[
            pl.BlockSpec((None, C, N), lambda b: (b, 0, 0)),
            pl.BlockSpec((None, C, N), lambda b: (b, 0, 0)),
            pl.BlockSpec((96, C), lambda b: (0, 0)),
            pl.BlockSpec((96, 1), lambda b: (0, 0)),
        ],
        out_specs=[
            pl.BlockSpec((None, N_HEADS, 16, N), lambda b: (b, 0, 0, 0)),
            pl.BlockSpec((None, N_HEADS, 16, N), lambda b: (b, 0, 0, 0)),
            pl.BlockSpec((None, N_HEADS, N, DH), lambda b: (b, 0, 0, 0)),
        ],
        out_shape=[
            jax.ShapeDtypeStruct((B, N_HEADS, 16, N), jnp.int32),
            jax.ShapeDtypeStruct((B, N_HEADS, 16, N), jnp.float32),
            jax.ShapeDtypeStruct((B, N_HEADS, N, DH), jnp.float32),
        ],
    )(query, key_feat, Wt, b_offp)


_CH = 128
_GROUPS = _CH // 16
_NCH = N // _CH


def _stage2_sc(table, idx_t, w_t):
    mesh = plsc.VectorSubcoreMesh(core_axis_name="c", subcore_axis_name="s")

    @functools.partial(
        pl.kernel,
        out_type=jax.ShapeDtypeStruct((B * N_HEADS, N * DH), jnp.float32),
        mesh=mesh,
        scratch_types=[
            pltpu.VMEM((N * DH,), jnp.float32),
            pltpu.VMEM((2, 16, _CH), jnp.int32),
            pltpu.VMEM((2, 16, _CH), jnp.float32),
            pltpu.VMEM((_CH * DH,), jnp.float32),
            pltpu.SemaphoreType.DMA,
            pltpu.SemaphoreType.DMA,
            pltpu.SemaphoreType.DMA,
        ],
        compiler_params=pltpu.CompilerParams(needs_layout_passes=False),
    )
    def k(table_hbm, idx_hbm, w_hbm, out_hbm,
          img_v, idx_v, w_v, out_v, sem_i, sem_w, sem_o):
        nc = 2
        wid = lax.axis_index("s") * nc + lax.axis_index("c")
        iota = lax.iota(jnp.int32, 16)

        def in_copies(img, t, s):
            p0 = t * _CH
            return (
                pltpu.make_async_copy(
                    idx_hbm.at[img, :, pl.ds(p0, _CH)], idx_v.at[s], sem_i),
                pltpu.make_async_copy(
                    w_hbm.at[img, :, pl.ds(p0, _CH)], w_v.at[s], sem_w),
            )

        def out_copy(img, t):
            return pltpu.make_async_copy(
                out_v, out_hbm.at[img, pl.ds(t * _CH * DH, _CH * DH)], sem_o)

        def img_body(u, _):
            img = wid * 2 + u
            pltpu.sync_copy(table_hbm.at[img], img_v)
            for cp in in_copies(img, 0, 0):
                cp.start()

            def pair_body(th, _, img=img):
                for s in range(2):
                    tt = th * 2 + s
                    for cp in in_copies(img, tt, s):
                        cp.wait()

                    @pl.when(tt + 1 < _NCH)
                    def _():
                        for cp in in_copies(img, tt + 1, 1 - s):
                            cp.start()

                    @pl.when(tt >= 1)
                    def _():
                        out_copy(img, tt - 1).wait()

                    for g in range(_GROUPS):
                        opix = (iota + g * 16) * DH
                        for half in range(2):
                            rows = [idx_v[s, j, pl.ds(g * 16, 16)] * DH
                                    for j in range(half * 8, half * 8 + 8)]
                            ws = [w_v[s, j, pl.ds(g * 16, 16)]
                                  for j in range(half * 8, half * 8 + 8)]

                            @plsc.parallel_loop(0, DH, unroll=4)
                            def _(t, rows=rows, ws=ws, opix=opix, half=half):
                                cvec = (t & ~15) + ((iota + t) & 15)
                                acc = ws[0] * plsc.load_gather(
                                    img_v, [rows[0] + cvec])
                                for j in range(1, 8):
                                    acc = acc + ws[j] * plsc.load_gather(
                                        img_v, [rows[j] + cvec])
                                if half == 0:
                                    plsc.store_scatter(
                                        out_v, [opix + cvec], acc)
                                else:
                                    plsc.addupdate_scatter(
                                        out_v, [opix + cvec], acc)

                    out_copy(img, tt).start()
                return 0

            lax.fori_loop(0, _NCH // 2, pair_body, 0)
            out_copy(img, _NCH - 1).wait()
            return 0

        lax.fori_loop(0, 2, img_body, 0)

    return k(table, idx_t, w_t)


def _stage3_body(f_ref, wp_ref, bp_ref, o_ref):
    acc = jnp.dot(f_ref[0], wp_ref[0], preferred_element_type=jnp.float32)
    for h in range(1, N_HEADS):
        acc = acc + jnp.dot(f_ref[h], wp_ref[h],
                            preferred_element_type=jnp.float32)
    o_ref[...] = acc + bp_ref[...]


def _stage3(feat_s, W_proj, b_proj):
    return pl.pallas_call(
        _stage3_body,
        grid=(B,),
        in_specs=[
            pl.BlockSpec((None, N_HEADS, N, DH), lambda b: (b, 0, 0, 0)),
            pl.BlockSpec((N_HEADS, DH, C), lambda b: (0, 0, 0)),
            pl.BlockSpec((1, C), lambda b: (0, 0)),
        ],
        out_specs=pl.BlockSpec((None, N, C), lambda b: (b, 0, 0)),
        out_shape=jax.ShapeDtypeStruct((B, N, C), jnp.float32),
    )(feat_s, W_proj.reshape(N_HEADS, DH, C), b_proj.reshape(1, C))


def kernel(query, key_feat, W_off, b_off, W_proj, b_proj):
    Wt = W_off.reshape(C, N_HEADS, N_POINTS, 3).transpose(0, 3, 1, 2)
    Wt = Wt.reshape(C, 96).T
    b_offp = b_off.reshape(N_HEADS, N_POINTS, 3).transpose(2, 0, 1).reshape(96, 1)

    qflat = query.reshape(B, C, N)
    kflat = key_feat.reshape(B, C, N)

    idx_t, w_t, key_t = _stage1(qflat, kflat, Wt, b_offp)

    table = key_t.reshape(B * N_HEADS, N * DH)
    idx_t = idx_t.reshape(B * N_HEADS, 16, N)
    w_t = w_t.reshape(B * N_HEADS, 16, N)

    feat_s = _stage2_sc(table, idx_t, w_t).reshape(B, N_HEADS, N, DH)

    return _stage3(feat_s, W_proj, b_proj)


# speedup vs baseline: 1.1836x; 1.1836x over previous
import functools
import numpy as np
import jax
import jax.numpy as jnp
from jax import lax
from jax.experimental import pallas as pl
from jax.experimental.pallas import tpu as pltpu
from jax.experimental.pallas import tpu_sc as plsc

N_HEADS = 8
N_POINTS = 4
B, C, H, W = 8, 768, 32, 32
DH = C // N_HEADS
N = H * W


def _stage1_body(q_ref, k_ref, wt_ref, b_ref, idx_ref, w_ref, kt_ref):
    o = jnp.dot(wt_ref[...], q_ref[...], preferred_element_type=jnp.float32,
                precision=jax.lax.Precision.HIGHEST)
    o = o + b_ref[...]
    ox = o[0:32].reshape(N_HEADS, N_POINTS, N)
    oy = o[32:64].reshape(N_HEADS, N_POINTS, N)
    s = o[64:96].reshape(N_HEADS, N_POINTS, N)

    m = jnp.max(s, axis=1, keepdims=True)
    e = jnp.exp(s - m)
    wsm = e / jnp.sum(e, axis=1, keepdims=True)

    n = jax.lax.broadcasted_iota(jnp.int32, (N_HEADS, N_POINTS, N), 2)
    xpix = (n & (W - 1)).astype(jnp.float32)
    ypix = (n >> 5).astype(jnp.float32)
    gx = -1.0 + xpix * (2.0 / (W - 1))
    gy = -1.0 + ypix * (2.0 / (H - 1))

    px = (gx + ox * 0.1 + 1.0) * ((W - 1) / 2.0)
    py = (gy + oy * 0.1 + 1.0) * ((H - 1) / 2.0)
    x0f = jnp.floor(px)
    y0f = jnp.floor(py)
    x0 = x0f.astype(jnp.int32)
    y0 = y0f.astype(jnp.int32)
    x1 = x0 + 1
    y1 = y0 + 1
    wx1 = px - x0f
    wx0 = 1.0 - wx1
    wy1 = py - y0f
    wy0 = 1.0 - wy1

    def corner(xc, yc, wxc, wyc):
        valid = ((xc >= 0) & (xc < W) & (yc >= 0) & (yc < H)).astype(jnp.float32)
        idx = jnp.clip(yc, 0, H - 1) * W + jnp.clip(xc, 0, W - 1)
        w = wsm * wyc * wxc * valid
        return idx, w

    i00, w00 = corner(x0, y0, wx0, wy0)
    i10, w10 = corner(x1, y0, wx1, wy0)
    i01, w01 = corner(x0, y1, wx0, wy1)
    i11, w11 = corner(x1, y1, wx1, wy1)
    idx_ref[...] = jnp.concatenate([i00, i10, i01, i11], axis=1)
    w_ref[...] = jnp.concatenate([w00, w10, w01, w11], axis=1)

    kt = k_ref[...].reshape(N_HEADS, DH, N)
    kt_ref[...] = jnp.swapaxes(kt, 1, 2)


def _stage1(query, key_feat, Wt, b_offp):
    return pl.pallas_call(
        _stage1_body,
        grid=(B,),
        in_specs=[
            pl.BlockSpec((None, C, N), lambda b: (b, 0, 0)),
            pl.BlockSpec((None, C, N), lambda b: (b, 0, 0)),
            pl.BlockSpec((96, C), lambda b: (0, 0)),
            pl.BlockSpec((96, 1), lambda b: (0, 0)),
        ],
        out_specs=[
            pl.BlockSpec((None, N_HEADS, 16, N), lambda b: (b, 0, 0, 0)),
            pl.BlockSpec((None, N_HEADS, 16, N), lambda b: (b, 0, 0, 0)),
            pl.BlockSpec((None, N_HEADS, N, DH), lambda b: (b, 0, 0, 0)),
        ],
        out_shape=[
            jax.ShapeDtypeStruct((B, N_HEADS, 16, N), jnp.int32),
            jax.ShapeDtypeStruct((B, N_HEADS, 16, N), jnp.float32),
            jax.ShapeDtypeStruct((B, N_HEADS, N, DH), jnp.float32),
        ],
    )(query, key_feat, Wt, b_offp)


_CH = 128
_GROUPS = _CH // 16
_NCH = N // _CH


def _stage2_sc(table, idx_t, w_t):
    mesh = plsc.VectorSubcoreMesh(core_axis_name="c", subcore_axis_name="s")

    @functools.partial(
        pl.kernel,
        out_type=jax.ShapeDtypeStruct((B * N_HEADS, N * DH), jnp.float32),
        mesh=mesh,
        scratch_types=[
            pltpu.VMEM((N * DH,), jnp.float32),
            pltpu.VMEM((2, 16, _CH), jnp.int32),
            pltpu.VMEM((2, 16, _CH), jnp.float32),
            pltpu.VMEM((_CH * DH,), jnp.float32),
            pltpu.SemaphoreType.DMA,
            pltpu.SemaphoreType.DMA,
            pltpu.SemaphoreType.DMA,
        ],
        compiler_params=pltpu.CompilerParams(needs_layout_passes=False),
    )
    def k(table_hbm, idx_hbm, w_hbm, out_hbm,
          img_v, idx_v, w_v, out_v, sem_i, sem_w, sem_o):
        nc = 2
        wid = lax.axis_index("s") * nc + lax.axis_index("c")
        iota = lax.iota(jnp.int32, 16)

        def in_copies(img, t, s):
            p0 = t * _CH
            return (
                pltpu.make_async_copy(
                    idx_hbm.at[img, :, pl.ds(p0, _CH)], idx_v.at[s], sem_i),
                pltpu.make_async_copy(
                    w_hbm.at[img, :, pl.ds(p0, _CH)], w_v.at[s], sem_w),
            )

        def out_copy(img, t):
            return pltpu.make_async_copy(
                out_v, out_hbm.at[img, pl.ds(t * _CH * DH, _CH * DH)], sem_o)

        def img_body(u, _):
            img = wid * 2 + u
            pltpu.sync_copy(table_hbm.at[img], img_v)
            for cp in in_copies(img, 0, 0):
                cp.start()

            def pair_body(th, _, img=img):
                for s in range(2):
                    tt = th * 2 + s
                    for cp in in_copies(img, tt, s):
                        cp.wait()

                    @pl.when(tt + 1 < _NCH)
                    def _():
                        for cp in in_copies(img, tt + 1, 1 - s):
                            cp.start()

                    @pl.when(tt >= 1)
                    def _():
                        out_copy(img, tt - 1).wait()

                    for g in range(_GROUPS):
                        opix = (iota + g * 16) * DH
                        for half in range(2):
                            rows = [idx_v[s, j, pl.ds(g * 16, 16)] * DH
                                    for j in range(half * 8, half * 8 + 8)]
                            ws = [w_v[s, j, pl.ds(g * 16, 16)]
                                  for j in range(half * 8, half * 8 + 8)]

                            @plsc.parallel_loop(0, DH, unroll=2)
                            def _(t, rows=rows, ws=ws, opix=opix, half=half):
                                cvec = (t & ~15) + ((iota + t) & 15)
                                acc = ws[0] * plsc.load_gather(
                                    img_v, [rows[0] + cvec])
                                for j in range(1, 8):
                                    acc = acc + ws[j] * plsc.load_gather(
                                        img_v, [rows[j] + cvec])
                                if half == 0:
                                    plsc.store_scatter(
                                        out_v, [opix + cvec], acc)
                                else:
                                    plsc.addupdate_scatter(
                                        out_v, [opix + cvec], acc)

                    out_copy(img, tt).start()
                return 0

            lax.fori_loop(0, _NCH // 2, pair_body, 0)
            out_copy(img, _NCH - 1).wait()
            return 0

        lax.fori_loop(0, 2, img_body, 0)

    return k(table, idx_t, w_t)


def _stage3_body(f_ref, wp_ref, bp_ref, o_ref):
    acc = jnp.dot(f_ref[0], wp_ref[0], preferred_element_type=jnp.float32)
    for h in range(1, N_HEADS):
        acc = acc + jnp.dot(f_ref[h], wp_ref[h],
                            preferred_element_type=jnp.float32)
    o_ref[...] = acc + bp_ref[...]


def _stage3(feat_s, W_proj, b_proj):
    return pl.pallas_call(
        _stage3_body,
        grid=(B,),
        in_specs=[
            pl.BlockSpec((None, N_HEADS, N, DH), lambda b: (b, 0, 0, 0)),
            pl.BlockSpec((N_HEADS, DH, C), lambda b: (0, 0, 0)),
            pl.BlockSpec((1, C), lambda b: (0, 0)),
        ],
        out_specs=pl.BlockSpec((None, N, C), lambda b: (b, 0, 0)),
        out_shape=jax.ShapeDtypeStruct((B, N, C), jnp.float32),
    )(feat_s, W_proj.reshape(N_HEADS, DH, C), b_proj.reshape(1, C))


def kernel(query, key_feat, W_off, b_off, W_proj, b_proj):
    Wt = W_off.reshape(C, N_HEADS, N_POINTS, 3).transpose(0, 3, 1, 2)
    Wt = Wt.reshape(C, 96).T
    b_offp = b_off.reshape(N_HEADS, N_POINTS, 3).transpose(2, 0, 1).reshape(96, 1)

    qflat = query.reshape(B, C, N)
    kflat = key_feat.reshape(B, C, N)

    idx_t, w_t, key_t = _stage1(qflat, kflat, Wt, b_offp)

    table = key_t.reshape(B * N_HEADS, N * DH)
    idx_t = idx_t.reshape(B * N_HEADS, 16, N)
    w_t = w_t.reshape(B * N_HEADS, 16, N)

    feat_s = _stage2_sc(table, idx_t, w_t).reshape(B, N_HEADS, N, DH)

    return _stage3(feat_s, W_proj, b_proj)
